# baseline (speedup 1.0000x reference)
import functools

import jax
import jax.numpy as jnp
from jax import lax
from jax.experimental import pallas as pl
from jax.experimental.pallas import tpu as pltpu
from jax.experimental.pallas import tpu_sc as plsc

_VOCAB = 1000000
_D = 128
_SCALE = float(_D) ** 0.5

_NC = 2
_NS = 16
_NW = _NC * _NS

_CHUNK = 128
_NBUF = 5
_LEAD = 4


def _make_kernel(n_rows):
    assert n_rows % (_NW * _CHUNK) == 0
    rows_per_w = n_rows // _NW
    n_chunks = rows_per_w // _CHUNK
    assert n_chunks % _NBUF == 0

    mesh = plsc.VectorSubcoreMesh(core_axis_name="c", subcore_axis_name="s")

    @functools.partial(
        pl.kernel,
        out_type=jax.ShapeDtypeStruct((n_rows, _D), jnp.float32),
        mesh=mesh,
        scratch_types=[
            pltpu.VMEM((n_chunks, _CHUNK), jnp.int32),
            pltpu.VMEM((_NBUF, _CHUNK, _D), jnp.float32),
            pltpu.SemaphoreType.DMA((_NBUF,)),
            pltpu.SemaphoreType.DMA((_NBUF,)),
        ],
    )
    def emb(idx_hbm, table_hbm, out_hbm, idx_v, rows, sem_g, sem_o):
        wid = lax.axis_index("s") * _NC + lax.axis_index("c")
        base = wid * rows_per_w

        pltpu.sync_copy(idx_hbm.at[wid], idx_v)

        for b in range(_LEAD):
            pltpu.async_copy(table_hbm.at[idx_v.at[b]], rows.at[b], sem_g.at[b])

        def outer(g, carry):
            for b in range(_NBUF):
                i = g * _NBUF + b
                rb = rows.at[b]

                j = i + _LEAD
                bj = (b + _LEAD) % _NBUF

                @pl.when(j < n_chunks)
                def _():
                    @pl.when(j >= _NBUF)
                    def _():
                        pltpu.make_async_copy(
                            rows.at[bj], out_hbm.at[pl.ds(0, _CHUNK)],
                            sem_o.at[bj]).wait()
                    pltpu.async_copy(
                        table_hbm.at[idx_v.at[j]], rows.at[bj], sem_g.at[bj])

                pltpu.make_async_copy(
                    table_hbm.at[idx_v.at[i]], rb, sem_g.at[b]).wait()

                @plsc.parallel_loop(0, _CHUNK, unroll=2)
                def scale_row(r):
                    for k in range(_D // 16):
                        sl = (r, pl.ds(k * 16, 16))
                        rb[sl] = rb[sl] * _SCALE

                pltpu.async_copy(
                    rb, out_hbm.at[pl.ds(base + i * _CHUNK, _CHUNK)],
                    sem_o.at[b])
            return carry

        lax.fori_loop(0, n_chunks // _NBUF, outer, 0)

        for b in range(_NBUF):
            pltpu.make_async_copy(
                rows.at[b], out_hbm.at[pl.ds(0, _CHUNK)], sem_o.at[b]).wait()

    return emb


@jax.jit
def kernel(x, table):
    batch, seq = x.shape
    n_rows = batch * seq
    rows_per_w = n_rows // _NW
    idx = x.reshape(_NW, rows_per_w // _CHUNK, _CHUNK).astype(jnp.int32)
    out = _make_kernel(n_rows)(idx, table)
    return out.reshape(batch, seq, _D)


# speedup vs baseline: 1.0099x; 1.0099x over previous
import functools

import jax
import jax.numpy as jnp
from jax import lax
from jax.experimental import pallas as pl
from jax.experimental.pallas import tpu as pltpu
from jax.experimental.pallas import tpu_sc as plsc

_VOCAB = 1000000
_D = 128
_SCALE = float(_D) ** 0.5

_NC = 2
_NS = 16
_NW = _NC * _NS

_CHUNK = 64
_NBUF = 8
_LEAD = 5


def _make_kernel(n_rows):
    assert n_rows % (_NW * _CHUNK) == 0
    rows_per_w = n_rows // _NW
    n_chunks = rows_per_w // _CHUNK
    assert n_chunks % _NBUF == 0

    mesh = plsc.VectorSubcoreMesh(core_axis_name="c", subcore_axis_name="s")

    @functools.partial(
        pl.kernel,
        out_type=jax.ShapeDtypeStruct((n_rows, _D), jnp.float32),
        mesh=mesh,
        scratch_types=[
            pltpu.VMEM((n_chunks, _CHUNK), jnp.int32),
            pltpu.VMEM((_NBUF, _CHUNK, _D), jnp.float32),
            pltpu.SemaphoreType.DMA((_NBUF,)),
            pltpu.SemaphoreType.DMA((_NBUF,)),
        ],
    )
    def emb(idx_hbm, table_hbm, out_hbm, idx_v, rows, sem_g, sem_o):
        wid = lax.axis_index("s") * _NC + lax.axis_index("c")
        base = wid * rows_per_w

        pltpu.sync_copy(idx_hbm.at[wid], idx_v)

        for b in range(_LEAD):
            pltpu.async_copy(table_hbm.at[idx_v.at[b]], rows.at[b], sem_g.at[b])

        def outer(g, carry):
            for b in range(_NBUF):
                i = g * _NBUF + b
                rb = rows.at[b]

                j = i + _LEAD
                bj = (b + _LEAD) % _NBUF

                @pl.when(j < n_chunks)
                def _():
                    @pl.when(j >= _NBUF)
                    def _():
                        pltpu.make_async_copy(
                            rows.at[bj], out_hbm.at[pl.ds(0, _CHUNK)],
                            sem_o.at[bj]).wait()
                    pltpu.async_copy(
                        table_hbm.at[idx_v.at[j]], rows.at[bj], sem_g.at[bj])

                pltpu.make_async_copy(
                    table_hbm.at[idx_v.at[i]], rb, sem_g.at[b]).wait()

                @plsc.parallel_loop(0, _CHUNK, unroll=2)
                def scale_row(r):
                    for k in range(_D // 16):
                        sl = (r, pl.ds(k * 16, 16))
                        rb[sl] = rb[sl] * _SCALE

                pltpu.async_copy(
                    rb, out_hbm.at[pl.ds(base + i * _CHUNK, _CHUNK)],
                    sem_o.at[b])
            return carry

        lax.fori_loop(0, n_chunks // _NBUF, outer, 0)

        for b in range(_NBUF):
            pltpu.make_async_copy(
                rows.at[b], out_hbm.at[pl.ds(0, _CHUNK)], sem_o.at[b]).wait()

    return emb


@jax.jit
def kernel(x, table):
    batch, seq = x.shape
    n_rows = batch * seq
    rows_per_w = n_rows // _NW
    idx = x.reshape(_NW, rows_per_w // _CHUNK, _CHUNK).astype(jnp.int32)
    out = _make_kernel(n_rows)(idx, table)
    return out.reshape(batch, seq, _D)


# speedup vs baseline: 1.0100x; 1.0002x over previous
import functools

import jax
import jax.numpy as jnp
from jax import lax
from jax.experimental import pallas as pl
from jax.experimental.pallas import tpu as pltpu
from jax.experimental.pallas import tpu_sc as plsc

_VOCAB = 1000000
_D = 128
_SCALE = float(_D) ** 0.5

_NC = 2
_NS = 16
_NW = _NC * _NS

_CHUNK = 128
_NBUF = 5
_LEAD = 3


def _make_kernel(n_rows):
    assert n_rows % (_NW * _CHUNK) == 0
    rows_per_w = n_rows // _NW
    n_chunks = rows_per_w // _CHUNK
    assert n_chunks % _NBUF == 0

    mesh = plsc.VectorSubcoreMesh(core_axis_name="c", subcore_axis_name="s")

    @functools.partial(
        pl.kernel,
        out_type=jax.ShapeDtypeStruct((n_rows, _D), jnp.float32),
        mesh=mesh,
        scratch_types=[
            pltpu.VMEM((n_chunks, _CHUNK), jnp.int32),
            pltpu.VMEM((_NBUF, _CHUNK, _D), jnp.float32),
            pltpu.SemaphoreType.DMA((_NBUF,)),
            pltpu.SemaphoreType.DMA((_NBUF,)),
        ],
    )
    def emb(idx_hbm, table_hbm, out_hbm, idx_v, rows, sem_g, sem_o):
        wid = lax.axis_index("s") * _NC + lax.axis_index("c")
        base = wid * rows_per_w

        pltpu.sync_copy(idx_hbm.at[wid], idx_v)

        for b in range(_LEAD):
            pltpu.async_copy(table_hbm.at[idx_v.at[b]], rows.at[b], sem_g.at[b])

        def outer(g, carry):
            for b in range(_NBUF):
                i = g * _NBUF + b
                rb = rows.at[b]

                j = i + _LEAD
                bj = (b + _LEAD) % _NBUF

                @pl.when(j < n_chunks)
                def _():
                    @pl.when(j >= _NBUF)
                    def _():
                        pltpu.make_async_copy(
                            rows.at[bj], out_hbm.at[pl.ds(0, _CHUNK)],
                            sem_o.at[bj]).wait()
                    pltpu.async_copy(
                        table_hbm.at[idx_v.at[j]], rows.at[bj], sem_g.at[bj])

                pltpu.make_async_copy(
                    table_hbm.at[idx_v.at[i]], rb, sem_g.at[b]).wait()

                @plsc.parallel_loop(0, _CHUNK, unroll=2)
                def scale_row(r):
                    for k in range(_D // 16):
                        sl = (r, pl.ds(k * 16, 16))
                        rb[sl] = rb[sl] * _SCALE

                pltpu.async_copy(
                    rb, out_hbm.at[pl.ds(base + i * _CHUNK, _CHUNK)],
                    sem_o.at[b])
            return carry

        lax.fori_loop(0, n_chunks // _NBUF, outer, 0)

        for b in range(_NBUF):
            pltpu.make_async_copy(
                rows.at[b], out_hbm.at[pl.ds(0, _CHUNK)], sem_o.at[b]).wait()

    return emb


@jax.jit
def kernel(x, table):
    batch, seq = x.shape
    n_rows = batch * seq
    rows_per_w = n_rows // _NW
    idx = x.reshape(_NW, rows_per_w // _CHUNK, _CHUNK).astype(jnp.int32)
    out = _make_kernel(n_rows)(idx, table)
    return out.reshape(batch, seq, _D)
